# Initial kernel scaffold; baseline (speedup 1.0000x reference)
import jax
import jax.numpy as jnp
from jax.experimental import pallas as pl


def kernel(h_author, h_paper, edge_writes, edge_written_by, params):
    raise NotImplementedError("write your pallas kernel here")


# speedup vs baseline: 1.8582x; 1.8582x over previous
import functools

import jax
import jax.numpy as jnp
from jax import lax
from jax.experimental import pallas as pl
from jax.experimental.pallas import tpu as pltpu
from jax.experimental.pallas import tpu_sc as plsc

N = 10000
E = 320000
NG = 10
GCH = 16
NCHUNK = NG * GCH
EPT = NCHUNK * 128
EPC = EPT * 16
NPAD = 10240
STRIPE = NPAD // 16
R = 1000
NB = N // R


def _mesh():
    return plsc.VectorSubcoreMesh(core_axis_name="c", subcore_axis_name="s",
                                  num_cores=1, num_subcores=16)


def _agg_body(x_hbm, src_hbm, dst_hbm, zrow_hbm, agg_out,
              sidx, didx, rows0, agg_sh, sem0):
    sid = lax.axis_index("s")
    base = sid * STRIPE

    pltpu.sync_copy(zrow_hbm, agg_sh.at[pl.ds(base, STRIPE)])
    plsc.subcore_barrier()

    def group(g, carry):
        pltpu.sync_copy(src_hbm.at[sid, g], sidx)
        pltpu.sync_copy(dst_hbm.at[sid, g], didx)

        def chunk(c, carry2):
            pltpu.async_copy(x_hbm.at[sidx.at[c]], rows0, sem0).wait()
            pltpu.sync_copy(rows0, agg_sh.at[didx.at[c]], add=True)
            return carry2

        lax.fori_loop(0, GCH, chunk, 0)
        return carry

    lax.fori_loop(0, NG, group, 0)
    plsc.subcore_barrier()

    for kk in range(5):
        off = base + kk * 128
        pltpu.sync_copy(agg_sh.at[pl.ds(off, 128)], rows0)
        pltpu.sync_copy(rows0, agg_out.at[pl.ds(off, 128)])


def _deg_body(dst_hbm, zrow_hbm, ones_hbm, deg_out, didx, ones_v, deg_sh):
    sid = lax.axis_index("s")
    base = sid * STRIPE
    pltpu.sync_copy(ones_hbm, ones_v)
    pltpu.sync_copy(zrow_hbm, deg_sh.at[pl.ds(base, STRIPE)])
    plsc.subcore_barrier()

    def group(g, carry):
        pltpu.sync_copy(dst_hbm.at[sid, g], didx)

        def chunk(c, carry2):
            pltpu.sync_copy(ones_v, deg_sh.at[didx.at[c]], add=True)
            return carry2

        lax.fori_loop(0, GCH, chunk, 0)
        return carry

    lax.fori_loop(0, NG, group, 0)
    plsc.subcore_barrier()
    for kk in range(5):
        off = base + kk * 128
        pltpu.sync_copy(deg_sh.at[pl.ds(off, 128)], ones_v)
        pltpu.sync_copy(ones_v, deg_out.at[pl.ds(off, 128)])


@functools.lru_cache(maxsize=None)
def _make_agg():
    scratch = [
        pltpu.VMEM((GCH, 128), jnp.int32),
        pltpu.VMEM((GCH, 128), jnp.int32),
        pltpu.VMEM((128, 128), jnp.float32),
        pltpu.VMEM_SHARED((NPAD, 128), jnp.float32),
        pltpu.SemaphoreType.DMA,
    ]
    return pl.kernel(_agg_body,
                     out_type=jax.ShapeDtypeStruct((NPAD, 128), jnp.float32),
                     mesh=_mesh(), scratch_types=scratch, name="sc_edge_agg")


@functools.lru_cache(maxsize=None)
def _make_deg():
    scratch = [
        pltpu.VMEM((GCH, 128), jnp.int32),
        pltpu.VMEM((128, 128), jnp.float32),
        pltpu.VMEM_SHARED((NPAD, 128), jnp.float32),
    ]
    return pl.kernel(_deg_body,
                     out_type=jax.ShapeDtypeStruct((NPAD, 128), jnp.float32),
                     mesh=_mesh(), scratch_types=scratch, name="sc_deg")


def _prep_edges(edge_writes, edge_written_by):
    sw = edge_writes[0].astype(jnp.int32)
    dw = edge_writes[1].astype(jnp.int32)
    sb = edge_written_by[0].astype(jnp.int32)
    db = edge_written_by[1].astype(jnp.int32)
    pad = EPC - E
    zpad = jnp.zeros((pad,), jnp.int32)
    dpad = jnp.full((pad,), N, jnp.int32)
    shape = (16, NG, GCH, 128)
    src0 = jnp.concatenate([sb + N, zpad]).reshape(shape)
    dst0 = jnp.concatenate([db, dpad]).reshape(shape)
    src1 = jnp.concatenate([sw, zpad]).reshape(shape)
    dst1 = jnp.concatenate([dw, dpad]).reshape(shape)
    return (src0, dst0), (src1, dst1)


def _prep_body(h_ref, agg_ref, deg_ref, Ws_ref, bs_ref, We_ref, be_ref,
               Wk_ref, bk_ref, Wa1_ref, ba1_ref, Wa2_ref,
               self_ref, d_ref, att_ref):
    i = pl.program_id(1)
    h = h_ref[0]
    self_b = jnp.dot(h, Ws_ref[0], preferred_element_type=jnp.float32) + bs_ref[0]
    deg = jnp.maximum(deg_ref[0][:, 0:1], 1.0)
    d_b = jnp.dot(agg_ref[0] / deg, We_ref[0],
                  preferred_element_type=jnp.float32) + be_ref[0]
    self_ref[0] = self_b
    d_ref[0] = d_b
    Wk = Wk_ref[0]
    bk = bk_ref[0]
    Wa1 = Wa1_ref[...]
    ba1 = ba1_ref[...]
    Wa2 = Wa2_ref[...]

    def score(x):
        k = jnp.dot(x, Wk, preferred_element_type=jnp.float32) + bk
        t = jnp.tanh(jnp.dot(k, Wa1, preferred_element_type=jnp.float32) + ba1)
        return jnp.sum(jnp.dot(t, Wa2, preferred_element_type=jnp.float32))

    s0 = score(self_b)
    s1 = score(d_b)

    @pl.when(i == 0)
    def _():
        att_ref[...] = jnp.zeros_like(att_ref)

    att_ref[0, 0] += s0
    att_ref[0, 1] += s1


def _comb_body(act, h_ref, self_ref, d_ref, att_ref, Wr_ref, br_ref, out_ref):
    s0 = jnp.max(att_ref[0, 0]) / N
    s1 = jnp.max(att_ref[0, 1]) / N
    mx = jnp.maximum(s0, s1)
    e0 = jnp.exp(s0 - mx)
    e1 = jnp.exp(s1 - mx)
    b0 = e0 / (e0 + e1)
    b1 = e1 / (e0 + e1)
    r = self_ref[0] * b0 + d_ref[0] * b1
    r = (r + jnp.dot(h_ref[0], Wr_ref[0], preferred_element_type=jnp.float32)
         + br_ref[0]) * 0.5
    if act:
        r = jnp.where(r > 0, r, jnp.exp(r) - 1.0)
    out_ref[0] = r


def _row_spec(last):
    return pl.BlockSpec((1, R, last), lambda t, i: (t, i, 0))


def _w_spec(a, b):
    return pl.BlockSpec((1, a, b), lambda t, i: (t, 0, 0))


_prep_call = pl.pallas_call(
    _prep_body,
    grid=(2, NB),
    in_specs=[
        _row_spec(128),
        _row_spec(128),
        _row_spec(128),
        _w_spec(128, 128),
        _w_spec(1, 128),
        _w_spec(128, 128),
        _w_spec(1, 128),
        _w_spec(128, 32),
        _w_spec(1, 32),
        pl.BlockSpec((32, 32), lambda t, i: (0, 0)),
        pl.BlockSpec((1, 32), lambda t, i: (0, 0)),
        pl.BlockSpec((32, 128), lambda t, i: (0, 0)),
    ],
    out_specs=[
        _row_spec(128),
        _row_spec(128),
        pl.BlockSpec((1, 2, 8, 128), lambda t, i: (t, 0, 0, 0)),
    ],
    out_shape=[
        jax.ShapeDtypeStruct((2, N, 128), jnp.float32),
        jax.ShapeDtypeStruct((2, N, 128), jnp.float32),
        jax.ShapeDtypeStruct((2, 2, 8, 128), jnp.float32),
    ],
)


def _make_comb(act):
    return pl.pallas_call(
        functools.partial(_comb_body, act),
        grid=(2, NB),
        in_specs=[
            _row_spec(128),
            _row_spec(128),
            _row_spec(128),
            pl.BlockSpec((1, 2, 8, 128), lambda t, i: (t, 0, 0, 0)),
            _w_spec(128, 128),
            _w_spec(1, 128),
        ],
        out_specs=_row_spec(128),
        out_shape=jax.ShapeDtypeStruct((2, N, 128), jnp.float32),
    )


_comb_act = _make_comb(True)
_comb_lin = _make_comb(False)


def _stack_params(p):
    Ws = jnp.stack([p["Ws_author"], p["Ws_paper"]])
    bs = jnp.stack([p["bs_author"], p["bs_paper"]])[:, None, :]
    We = jnp.stack([p["We_written_by"], p["We_writes"]])
    be = jnp.stack([p["be_written_by"], p["be_writes"]])[:, None, :]
    Wk = jnp.stack([p["Wk_author"], p["Wk_paper"]])
    bk = jnp.stack([p["bk_author"], p["bk_paper"]])[:, None, :]
    Wr = jnp.stack([p["Wr_author"], p["Wr_paper"]])
    br = jnp.stack([p["br_author"], p["br_paper"]])[:, None, :]
    Wa2 = jnp.pad(p["Wa2"], ((0, 0), (0, 127)))
    return Ws, bs, We, be, Wk, bk, Wr, br, p["Wa1"], p["ba1"][None, :], Wa2


def kernel(h_author, h_paper, edge_writes, edge_written_by, params):
    e0, e1 = _prep_edges(edge_writes, edge_written_by)
    zrow = jnp.zeros((STRIPE, 128), jnp.float32)
    ones = jnp.ones((128, 128), jnp.float32)

    deg0 = _make_deg()(e0[1], zrow, ones)
    deg1 = _make_deg()(e1[1], zrow, ones)
    deg = jnp.stack([deg0[:N], deg1[:N]])

    h = jnp.stack([h_author, h_paper])
    for l, p in enumerate(params):
        x_cat = h.reshape(2 * N, 128)
        agg0 = _make_agg()(x_cat, e0[0], e0[1], zrow)
        agg1 = _make_agg()(x_cat, e1[0], e1[1], zrow)
        agg = jnp.stack([agg0[:N], agg1[:N]])
        Ws, bs, We, be, Wk, bk, Wr, br, Wa1, ba1, Wa2 = _stack_params(p)
        selfx, dx, att = _prep_call(h, agg, deg,
                                    Ws, bs, We, be, Wk, bk, Wa1, ba1, Wa2)
        comb = _comb_act if l == 0 else _comb_lin
        h = comb(h, selfx, dx, att, Wr, br)
    return h[0], h[1]


# speedup vs baseline: 2.6517x; 1.4270x over previous
import functools

import jax
import jax.numpy as jnp
from jax import lax
from jax.experimental import pallas as pl
from jax.experimental.pallas import tpu as pltpu
from jax.experimental.pallas import tpu_sc as plsc

N = 10000
E = 320000
NG = 10
GCH = 16
NCHUNK = NG * GCH
EPT = NCHUNK * 128
EPC = EPT * 16
NPAD = 10240
STRIPE = NPAD // 16
R = 1000
NB = N // R


def _mesh():
    return plsc.VectorSubcoreMesh(core_axis_name="c", subcore_axis_name="s",
                                  num_cores=2, num_subcores=16)


def _agg_body(x_hbm, src_hbm, dst_hbm, zrow_hbm, agg_out,
              sidx, didx, rows0, agg_sh, sem0):
    cid = lax.axis_index("c")
    sid = lax.axis_index("s")
    base = sid * STRIPE

    pltpu.sync_copy(zrow_hbm, agg_sh.at[pl.ds(base, STRIPE)])
    plsc.subcore_barrier()

    def group(g, carry):
        pltpu.sync_copy(src_hbm.at[cid, sid, g], sidx)
        pltpu.sync_copy(dst_hbm.at[cid, sid, g], didx)

        def chunk(c, carry2):
            pltpu.async_copy(x_hbm.at[sidx.at[c]], rows0, sem0).wait()
            pltpu.sync_copy(rows0, agg_sh.at[didx.at[c]], add=True)
            return carry2

        lax.fori_loop(0, GCH, chunk, 0)
        return carry

    lax.fori_loop(0, NG, group, 0)
    plsc.subcore_barrier()

    for kk in range(5):
        off = base + kk * 128
        pltpu.sync_copy(agg_sh.at[pl.ds(off, 128)], rows0)
        pltpu.sync_copy(rows0, agg_out.at[cid, pl.ds(off, 128)])


def _deg_body(dst_hbm, zrow_hbm, ones_hbm, deg_out, didx, ones_v, deg_sh):
    cid = lax.axis_index("c")
    sid = lax.axis_index("s")
    base = sid * STRIPE
    pltpu.sync_copy(ones_hbm, ones_v)
    pltpu.sync_copy(zrow_hbm, deg_sh.at[pl.ds(base, STRIPE)])
    plsc.subcore_barrier()

    def group(g, carry):
        pltpu.sync_copy(dst_hbm.at[cid, sid, g], didx)

        def chunk(c, carry2):
            pltpu.sync_copy(ones_v, deg_sh.at[didx.at[c]], add=True)
            return carry2

        lax.fori_loop(0, GCH, chunk, 0)
        return carry

    lax.fori_loop(0, NG, group, 0)
    plsc.subcore_barrier()
    for kk in range(5):
        off = base + kk * 128
        pltpu.sync_copy(deg_sh.at[pl.ds(off, 128)], ones_v)
        pltpu.sync_copy(ones_v, deg_out.at[cid, pl.ds(off, 128)])


@functools.lru_cache(maxsize=None)
def _make_agg():
    scratch = [
        pltpu.VMEM((GCH, 128), jnp.int32),
        pltpu.VMEM((GCH, 128), jnp.int32),
        pltpu.VMEM((128, 128), jnp.float32),
        pltpu.VMEM_SHARED((NPAD, 128), jnp.float32),
        pltpu.SemaphoreType.DMA,
    ]
    return pl.kernel(_agg_body,
                     out_type=jax.ShapeDtypeStruct((2, NPAD, 128), jnp.float32),
                     mesh=_mesh(), scratch_types=scratch, name="sc_edge_agg")


@functools.lru_cache(maxsize=None)
def _make_deg():
    scratch = [
        pltpu.VMEM((GCH, 128), jnp.int32),
        pltpu.VMEM((128, 128), jnp.float32),
        pltpu.VMEM_SHARED((NPAD, 128), jnp.float32),
    ]
    return pl.kernel(_deg_body,
                     out_type=jax.ShapeDtypeStruct((2, NPAD, 128), jnp.float32),
                     mesh=_mesh(), scratch_types=scratch, name="sc_deg")


def _prep_edges(edge_writes, edge_written_by):
    sw = edge_writes[0].astype(jnp.int32)
    dw = edge_writes[1].astype(jnp.int32)
    sb = edge_written_by[0].astype(jnp.int32)
    db = edge_written_by[1].astype(jnp.int32)
    pad = EPC - E
    zpad = jnp.zeros((pad,), jnp.int32)
    dpad = jnp.full((pad,), N, jnp.int32)
    shape = (16, NG, GCH, 128)
    src0 = jnp.concatenate([sb + N, zpad]).reshape(shape)
    dst0 = jnp.concatenate([db, dpad]).reshape(shape)
    src1 = jnp.concatenate([sw, zpad]).reshape(shape)
    dst1 = jnp.concatenate([dw, dpad]).reshape(shape)
    return jnp.stack([src0, src1]), jnp.stack([dst0, dst1])


def _prep_body(h_ref, agg_ref, deg_ref, Ws_ref, bs_ref, We_ref, be_ref,
               Wk_ref, bk_ref, Wa1_ref, ba1_ref, Wa2_ref,
               self_ref, d_ref, att_ref):
    i = pl.program_id(1)
    h = h_ref[0]
    self_b = jnp.dot(h, Ws_ref[0], preferred_element_type=jnp.float32) + bs_ref[0]
    deg = jnp.maximum(deg_ref[0][:, 0:1], 1.0)
    d_b = jnp.dot(agg_ref[0] / deg, We_ref[0],
                  preferred_element_type=jnp.float32) + be_ref[0]
    self_ref[0] = self_b
    d_ref[0] = d_b
    Wk = Wk_ref[0]
    bk = bk_ref[0]
    Wa1 = Wa1_ref[...]
    ba1 = ba1_ref[...]
    Wa2 = Wa2_ref[...]

    def score(x):
        k = jnp.dot(x, Wk, preferred_element_type=jnp.float32) + bk
        t = jnp.tanh(jnp.dot(k, Wa1, preferred_element_type=jnp.float32) + ba1)
        return jnp.sum(jnp.dot(t, Wa2, preferred_element_type=jnp.float32))

    s0 = score(self_b)
    s1 = score(d_b)

    @pl.when(i == 0)
    def _():
        att_ref[...] = jnp.zeros_like(att_ref)

    att_ref[0, 0] += s0
    att_ref[0, 1] += s1


def _comb_body(act, h_ref, self_ref, d_ref, att_ref, Wr_ref, br_ref, out_ref):
    s0 = jnp.max(att_ref[0, 0]) / N
    s1 = jnp.max(att_ref[0, 1]) / N
    mx = jnp.maximum(s0, s1)
    e0 = jnp.exp(s0 - mx)
    e1 = jnp.exp(s1 - mx)
    b0 = e0 / (e0 + e1)
    b1 = e1 / (e0 + e1)
    r = self_ref[0] * b0 + d_ref[0] * b1
    r = (r + jnp.dot(h_ref[0], Wr_ref[0], preferred_element_type=jnp.float32)
         + br_ref[0]) * 0.5
    if act:
        r = jnp.where(r > 0, r, jnp.exp(r) - 1.0)
    out_ref[0] = r


def _row_spec(last):
    return pl.BlockSpec((1, R, last), lambda t, i: (t, i, 0))


def _w_spec(a, b):
    return pl.BlockSpec((1, a, b), lambda t, i: (t, 0, 0))


_prep_call = pl.pallas_call(
    _prep_body,
    grid=(2, NB),
    in_specs=[
        _row_spec(128),
        _row_spec(128),
        _row_spec(128),
        _w_spec(128, 128),
        _w_spec(1, 128),
        _w_spec(128, 128),
        _w_spec(1, 128),
        _w_spec(128, 32),
        _w_spec(1, 32),
        pl.BlockSpec((32, 32), lambda t, i: (0, 0)),
        pl.BlockSpec((1, 32), lambda t, i: (0, 0)),
        pl.BlockSpec((32, 128), lambda t, i: (0, 0)),
    ],
    out_specs=[
        _row_spec(128),
        _row_spec(128),
        pl.BlockSpec((1, 2, 8, 128), lambda t, i: (t, 0, 0, 0)),
    ],
    out_shape=[
        jax.ShapeDtypeStruct((2, N, 128), jnp.float32),
        jax.ShapeDtypeStruct((2, N, 128), jnp.float32),
        jax.ShapeDtypeStruct((2, 2, 8, 128), jnp.float32),
    ],
)


def _make_comb(act):
    return pl.pallas_call(
        functools.partial(_comb_body, act),
        grid=(2, NB),
        in_specs=[
            _row_spec(128),
            _row_spec(128),
            _row_spec(128),
            pl.BlockSpec((1, 2, 8, 128), lambda t, i: (t, 0, 0, 0)),
            _w_spec(128, 128),
            _w_spec(1, 128),
        ],
        out_specs=_row_spec(128),
        out_shape=jax.ShapeDtypeStruct((2, N, 128), jnp.float32),
    )


_comb_act = _make_comb(True)
_comb_lin = _make_comb(False)


def _stack_params(p):
    Ws = jnp.stack([p["Ws_author"], p["Ws_paper"]])
    bs = jnp.stack([p["bs_author"], p["bs_paper"]])[:, None, :]
    We = jnp.stack([p["We_written_by"], p["We_writes"]])
    be = jnp.stack([p["be_written_by"], p["be_writes"]])[:, None, :]
    Wk = jnp.stack([p["Wk_author"], p["Wk_paper"]])
    bk = jnp.stack([p["bk_author"], p["bk_paper"]])[:, None, :]
    Wr = jnp.stack([p["Wr_author"], p["Wr_paper"]])
    br = jnp.stack([p["br_author"], p["br_paper"]])[:, None, :]
    Wa2 = jnp.pad(p["Wa2"], ((0, 0), (0, 127)))
    return Ws, bs, We, be, Wk, bk, Wr, br, p["Wa1"], p["ba1"][None, :], Wa2


def kernel(h_author, h_paper, edge_writes, edge_written_by, params):
    esrc, edst = _prep_edges(edge_writes, edge_written_by)
    zrow = jnp.zeros((STRIPE, 128), jnp.float32)
    ones = jnp.ones((128, 128), jnp.float32)

    deg = _make_deg()(edst, zrow, ones)[:, :N]

    h = jnp.stack([h_author, h_paper])
    for l, p in enumerate(params):
        x_cat = h.reshape(2 * N, 128)
        agg = _make_agg()(x_cat, esrc, edst, zrow)[:, :N]
        Ws, bs, We, be, Wk, bk, Wr, br, Wa1, ba1, Wa2 = _stack_params(p)
        selfx, dx, att = _prep_call(h, agg, deg,
                                    Ws, bs, We, be, Wk, bk, Wa1, ba1, Wa2)
        comb = _comb_act if l == 0 else _comb_lin
        h = comb(h, selfx, dx, att, Wr, br)
    return h[0], h[1]


# speedup vs baseline: 2.8735x; 1.0836x over previous
import functools

import jax
import jax.numpy as jnp
from jax import lax
from jax.experimental import pallas as pl
from jax.experimental.pallas import tpu as pltpu
from jax.experimental.pallas import tpu_sc as plsc

N = 10000
E = 320000
NG = 10
GCH = 16
NCHUNK = NG * GCH
EPT = NCHUNK * 128
EPC = EPT * 16
NPAD = 10240
STRIPE = NPAD // 16
R = 1000
NB = N // R


def _mesh():
    return plsc.VectorSubcoreMesh(core_axis_name="c", subcore_axis_name="s",
                                  num_cores=2, num_subcores=16)


def _agg_body(x_hbm, src_hbm, dst_hbm, zrow_hbm, agg_out,
              sidx, didx, rows0, rows1, agg_sh, sem0, sem1):
    cid = lax.axis_index("c")
    sid = lax.axis_index("s")
    base = sid * STRIPE

    pltpu.sync_copy(zrow_hbm, agg_sh.at[pl.ds(base, STRIPE)])
    plsc.subcore_barrier()

    def group(g, carry):
        pltpu.sync_copy(src_hbm.at[cid, sid, pl.ds(g * GCH, GCH)], sidx)
        pltpu.sync_copy(dst_hbm.at[cid, sid, pl.ds(g * GCH, GCH)], didx)
        pltpu.async_copy(x_hbm.at[sidx.at[0]], rows0, sem0)

        def pair(k, carry2):
            c0 = 2 * k
            pltpu.async_copy(x_hbm.at[sidx.at[c0 + 1]], rows1, sem1)
            pltpu.make_async_copy(x_hbm.at[sidx.at[c0]], rows0, sem0).wait()
            pltpu.sync_copy(rows0, agg_sh.at[didx.at[c0]], add=True)
            pltpu.async_copy(x_hbm.at[sidx.at[c0 + 2]], rows0, sem0)
            pltpu.make_async_copy(x_hbm.at[sidx.at[c0 + 1]], rows1, sem1).wait()
            pltpu.sync_copy(rows1, agg_sh.at[didx.at[c0 + 1]], add=True)
            return carry2

        lax.fori_loop(0, GCH // 2 - 1, pair, 0)
        last = GCH - 2
        pltpu.async_copy(x_hbm.at[sidx.at[last + 1]], rows1, sem1)
        pltpu.make_async_copy(x_hbm.at[sidx.at[last]], rows0, sem0).wait()
        pltpu.sync_copy(rows0, agg_sh.at[didx.at[last]], add=True)
        pltpu.make_async_copy(x_hbm.at[sidx.at[last + 1]], rows1, sem1).wait()
        pltpu.sync_copy(rows1, agg_sh.at[didx.at[last + 1]], add=True)
        return carry

    lax.fori_loop(0, NG, group, 0)
    plsc.subcore_barrier()

    for kk in range(5):
        off = base + kk * 128
        pltpu.sync_copy(agg_sh.at[pl.ds(off, 128)], rows0)
        pltpu.sync_copy(rows0, agg_out.at[cid, pl.ds(off, 128)])


def _deg_body(dst_hbm, zrow_hbm, ones_hbm, deg_out, didx, ones_v, deg_sh):
    cid = lax.axis_index("c")
    sid = lax.axis_index("s")
    base = sid * STRIPE
    pltpu.sync_copy(ones_hbm, ones_v)
    pltpu.sync_copy(dst_hbm.at[cid, sid], didx)
    pltpu.sync_copy(zrow_hbm, deg_sh.at[pl.ds(base, STRIPE)])
    plsc.subcore_barrier()

    def chunk(c, carry):
        pltpu.sync_copy(ones_v, deg_sh.at[didx.at[c]], add=True)
        return carry

    lax.fori_loop(0, NCHUNK, chunk, 0)
    plsc.subcore_barrier()
    for kk in range(5):
        off = base + kk * 128
        pltpu.sync_copy(deg_sh.at[pl.ds(off, 128)], ones_v)
        pltpu.sync_copy(ones_v, deg_out.at[cid, pl.ds(off, 128)])


@functools.lru_cache(maxsize=None)
def _make_agg():
    scratch = [
        pltpu.VMEM((GCH, 128), jnp.int32),
        pltpu.VMEM((GCH, 128), jnp.int32),
        pltpu.VMEM((128, 128), jnp.float32),
        pltpu.VMEM((128, 128), jnp.float32),
        pltpu.VMEM_SHARED((NPAD, 128), jnp.float32),
        pltpu.SemaphoreType.DMA,
        pltpu.SemaphoreType.DMA,
    ]
    return pl.kernel(_agg_body,
                     out_type=jax.ShapeDtypeStruct((2, NPAD, 128), jnp.float32),
                     mesh=_mesh(), scratch_types=scratch, name="sc_edge_agg")


@functools.lru_cache(maxsize=None)
def _make_deg():
    scratch = [
        pltpu.VMEM((NCHUNK, 128), jnp.int32),
        pltpu.VMEM((128, 128), jnp.float32),
        pltpu.VMEM_SHARED((NPAD, 128), jnp.float32),
    ]
    return pl.kernel(_deg_body,
                     out_type=jax.ShapeDtypeStruct((2, NPAD, 128), jnp.float32),
                     mesh=_mesh(), scratch_types=scratch, name="sc_deg")


def _prep_edges(edge_writes, edge_written_by):
    sw = edge_writes[0].astype(jnp.int32)
    dw = edge_writes[1].astype(jnp.int32)
    sb = edge_written_by[0].astype(jnp.int32)
    db = edge_written_by[1].astype(jnp.int32)
    pad = EPC - E
    zpad = jnp.zeros((pad,), jnp.int32)
    dpad = jnp.full((pad,), N, jnp.int32)
    shape = (16, NCHUNK, 128)
    src0 = jnp.concatenate([sb + N, zpad]).reshape(shape)
    dst0 = jnp.concatenate([db, dpad]).reshape(shape)
    src1 = jnp.concatenate([sw, zpad]).reshape(shape)
    dst1 = jnp.concatenate([dw, dpad]).reshape(shape)
    return jnp.stack([src0, src1]), jnp.stack([dst0, dst1])


def _prep_body(h_ref, agg_ref, deg_ref, Ws_ref, bs_ref, We_ref, be_ref,
               Wk_ref, bk_ref, Wa1_ref, ba1_ref, Wa2_ref,
               att_ref):
    i = pl.program_id(1)
    h = h_ref[0]
    self_b = jnp.dot(h, Ws_ref[0], preferred_element_type=jnp.float32) + bs_ref[0]
    deg = jnp.maximum(deg_ref[0][:, 0:1], 1.0)
    d_b = jnp.dot(agg_ref[0] / deg, We_ref[0],
                  preferred_element_type=jnp.float32) + be_ref[0]
    Wk = Wk_ref[0]
    bk = bk_ref[0]
    Wa1 = Wa1_ref[...]
    ba1 = ba1_ref[...]
    Wa2 = Wa2_ref[...]

    def score(x):
        k = jnp.dot(x, Wk, preferred_element_type=jnp.float32) + bk
        t = jnp.tanh(jnp.dot(k, Wa1, preferred_element_type=jnp.float32) + ba1)
        return jnp.sum(jnp.dot(t, Wa2, preferred_element_type=jnp.float32))

    s0 = score(self_b)
    s1 = score(d_b)

    @pl.when(i == 0)
    def _():
        att_ref[...] = jnp.zeros_like(att_ref)

    att_ref[0, 0] += s0
    att_ref[0, 1] += s1


def _comb_body(act, h_ref, agg_ref, deg_ref, att_ref, Ws_ref, bs_ref,
               We_ref, be_ref, Wr_ref, br_ref, out_ref):
    s0 = jnp.max(att_ref[0, 0]) / N
    s1 = jnp.max(att_ref[0, 1]) / N
    mx = jnp.maximum(s0, s1)
    e0 = jnp.exp(s0 - mx)
    e1 = jnp.exp(s1 - mx)
    b0 = e0 / (e0 + e1)
    b1 = e1 / (e0 + e1)
    h = h_ref[0]
    self_b = jnp.dot(h, Ws_ref[0], preferred_element_type=jnp.float32) + bs_ref[0]
    deg = jnp.maximum(deg_ref[0][:, 0:1], 1.0)
    d_b = jnp.dot(agg_ref[0] / deg, We_ref[0],
                  preferred_element_type=jnp.float32) + be_ref[0]
    r = self_b * b0 + d_b * b1
    r = (r + jnp.dot(h, Wr_ref[0], preferred_element_type=jnp.float32)
         + br_ref[0]) * 0.5
    if act:
        r = jnp.where(r > 0, r, jnp.exp(r) - 1.0)
    out_ref[0] = r


def _row_spec(last):
    return pl.BlockSpec((1, R, last), lambda t, i: (t, i, 0))


def _w_spec(a, b):
    return pl.BlockSpec((1, a, b), lambda t, i: (t, 0, 0))


_prep_call = pl.pallas_call(
    _prep_body,
    grid=(2, NB),
    in_specs=[
        _row_spec(128),
        _row_spec(128),
        _row_spec(128),
        _w_spec(128, 128),
        _w_spec(1, 128),
        _w_spec(128, 128),
        _w_spec(1, 128),
        _w_spec(128, 32),
        _w_spec(1, 32),
        pl.BlockSpec((32, 32), lambda t, i: (0, 0)),
        pl.BlockSpec((1, 32), lambda t, i: (0, 0)),
        pl.BlockSpec((32, 128), lambda t, i: (0, 0)),
    ],
    out_specs=pl.BlockSpec((1, 2, 8, 128), lambda t, i: (t, 0, 0, 0)),
    out_shape=jax.ShapeDtypeStruct((2, 2, 8, 128), jnp.float32),
)


def _make_comb(act):
    return pl.pallas_call(
        functools.partial(_comb_body, act),
        grid=(2, NB),
        in_specs=[
            _row_spec(128),
            _row_spec(128),
            _row_spec(128),
            pl.BlockSpec((1, 2, 8, 128), lambda t, i: (t, 0, 0, 0)),
            _w_spec(128, 128),
            _w_spec(1, 128),
            _w_spec(128, 128),
            _w_spec(1, 128),
            _w_spec(128, 128),
            _w_spec(1, 128),
        ],
        out_specs=_row_spec(128),
        out_shape=jax.ShapeDtypeStruct((2, N, 128), jnp.float32),
    )


_comb_act = _make_comb(True)
_comb_lin = _make_comb(False)


def _stack_params(p):
    Ws = jnp.stack([p["Ws_author"], p["Ws_paper"]])
    bs = jnp.stack([p["bs_author"], p["bs_paper"]])[:, None, :]
    We = jnp.stack([p["We_written_by"], p["We_writes"]])
    be = jnp.stack([p["be_written_by"], p["be_writes"]])[:, None, :]
    Wk = jnp.stack([p["Wk_author"], p["Wk_paper"]])
    bk = jnp.stack([p["bk_author"], p["bk_paper"]])[:, None, :]
    Wr = jnp.stack([p["Wr_author"], p["Wr_paper"]])
    br = jnp.stack([p["br_author"], p["br_paper"]])[:, None, :]
    Wa2 = jnp.pad(p["Wa2"], ((0, 0), (0, 127)))
    return Ws, bs, We, be, Wk, bk, Wr, br, p["Wa1"], p["ba1"][None, :], Wa2


def kernel(h_author, h_paper, edge_writes, edge_written_by, params):
    esrc, edst = _prep_edges(edge_writes, edge_written_by)
    zrow = jnp.zeros((STRIPE, 128), jnp.float32)
    ones = jnp.ones((128, 128), jnp.float32)

    deg = _make_deg()(edst, zrow, ones)[:, :N]

    h = jnp.stack([h_author, h_paper])
    for l, p in enumerate(params):
        x_cat = h.reshape(2 * N, 128)
        agg = _make_agg()(x_cat, esrc, edst, zrow)[:, :N]
        Ws, bs, We, be, Wk, bk, Wr, br, Wa1, ba1, Wa2 = _stack_params(p)
        att = _prep_call(h, agg, deg, Ws, bs, We, be, Wk, bk, Wa1, ba1, Wa2)
        comb = _comb_act if l == 0 else _comb_lin
        h = comb(h, agg, deg, att, Ws, bs, We, be, Wr, br)
    return h[0], h[1]
